# baseline (speedup 1.0000x reference)
import dataclasses
import functools

import jax
import jax.numpy as jnp
from jax import lax
from jax.experimental import pallas as pl
from jax.experimental.pallas import tpu as pltpu
from jax.experimental.pallas import tpu_sc as plsc

_B, _L, _D = 4096, 200, 64
_NC, _NS = 2, 16
_NW = _NC * _NS
_NCH = _B // _NW
_NT, _NLOC, _NTIME, _NMETH = 9, 1000, 365, 100
_SPLITS = (0, 128)


def _build_sc_kernel():
    mesh = plsc.VectorSubcoreMesh(core_axis_name="c", subcore_axis_name="s")
    cp = pltpu.CompilerParams()
    if "needs_layout_passes" in pltpu.CompilerParams.__dataclass_fields__:
        cp = dataclasses.replace(cp, needs_layout_passes=False)
    if "use_tc_tiling_on_sc" in pltpu.CompilerParams.__dataclass_fields__:
        cp = dataclasses.replace(cp, use_tc_tiling_on_sc=False)

    scratch = []
    for _ in range(2):
        scratch += [pltpu.VMEM((5, _L), jnp.int32)]
        scratch += [pltpu.VMEM((_L,), jnp.float32)]
        scratch += [pltpu.VMEM((_L, _D), jnp.float32)]
    scratch += [
        pltpu.VMEM((_NT, _D), jnp.float32),
        pltpu.VMEM((_NLOC, _D), jnp.float32),
        pltpu.VMEM((_NTIME, _D), jnp.float32),
        pltpu.VMEM((_NMETH, _D), jnp.float32),
        pltpu.VMEM((_D,), jnp.float32),
        pltpu.VMEM((_D,), jnp.float32),
    ]
    scratch += [pltpu.SemaphoreType.DMA] * 6

    @functools.partial(
        pl.kernel,
        compiler_params=cp,
        out_type=jax.ShapeDtypeStruct((_B, _L, _D), jnp.float32),
        mesh=mesh,
        scratch_types=scratch,
    )
    def k(ti_hbm, li_hbm, mi_hbm, ai_hbm, ei_hbm, q_hbm,
          ttab, ltab, titab, mtab, etab, wq_hbm, bq_hbm, out_hbm, *scr):
        stk = [scr[0], scr[3]]
        qv = [scr[1], scr[4]]
        ab = [scr[2], scr[5]]
        tres, lres, mres, eres, wq_v, bq_v = scr[6:12]
        sem_idx, sem_g, sem_out = scr[12:14], scr[14:16], scr[16:18]

        idx_hbm = [ti_hbm, li_hbm, mi_hbm, ai_hbm, ei_hbm]

        wid = lax.axis_index("s") * _NC + lax.axis_index("c")
        pltpu.sync_copy(ttab, tres)
        pltpu.sync_copy(ltab, lres)
        pltpu.sync_copy(titab, mres)
        pltpu.sync_copy(etab, eres)
        pltpu.sync_copy(wq_hbm, wq_v)
        pltpu.sync_copy(bq_hbm, bq_v)
        wq = [wq_v[pl.ds(i * 16, 16)] for i in range(4)]
        bq = [bq_v[pl.ds(i * 16, 16)] for i in range(4)]

        @pl.loop(0, _NT)
        def _(r):
            for dd in range(4):
                sl = pl.ds(dd * 16, 16)
                tres[r, sl] = tres[r, sl] + bq[dd]

        def fire_idx(j, s):
            b = wid * _NCH + j
            for r, hbm in enumerate(idx_hbm):
                pltpu.async_copy(hbm.at[b], stk[s].at[r], sem_idx[s])
            pltpu.async_copy(q_hbm.at[b], qv[s], sem_idx[s])

        def wait_idx(s):
            pltpu.make_async_copy(ti_hbm.at[pl.ds(0, 5)], stk[s],
                                  sem_idx[s]).wait()
            pltpu.make_async_copy(q_hbm.at[0], qv[s], sem_idx[s]).wait()

        def fire_gather(s):
            iv = stk[s].at[3]
            bounds = _SPLITS + (_L,)
            for h, r0 in enumerate(_SPLITS):
                n = bounds[h + 1] - r0
                pltpu.async_copy(mtab.at[iv.at[pl.ds(r0, n)]],
                                 ab[s].at[pl.ds(r0, n)], sem_g[s])

        def wait_gather(s):
            pltpu.make_async_copy(mtab.at[stk[s].at[3]], ab[s],
                                  sem_g[s]).wait()

        def fire_out(j, s):
            b = wid * _NCH + j
            pltpu.async_copy(ab[s], out_hbm.at[b], sem_out[s])

        def wait_out(s):
            pltpu.make_async_copy(ab[s], out_hbm.at[0], sem_out[s]).wait()

        def compute(s):
            sk = stk[s]
            qvv = qv[s]
            acc = ab[s]

            def do_group(gb, kks):
                sl16 = pl.ds(gb, 16)
                tvec = sk[0, sl16]
                lvec = sk[1, sl16]
                mvec = sk[2, sl16]
                evec = sk[4, sl16]
                qvec = qvv[sl16]
                for kk in kks:
                    t = gb + kk
                    it, il = tvec[kk], lvec[kk]
                    im, ie = mvec[kk], evec[kk]
                    q = lax.broadcast(qvec[kk], (16,))
                    for dd in range(4):
                        sl = pl.ds(dd * 16, 16)
                        s1 = acc[t, sl] + tres[it, sl]
                        s2 = lres[il, sl] + mres[im, sl]
                        s3 = eres[ie, sl] + q * wq[dd]
                        acc[t, sl] = (s1 + s2) + s3

            @pl.loop(0, (_L - 8) // 16)
            def _(g):
                do_group(g * 16, range(16))

            do_group(_L - 16, range(8, 16))

        def phase(j, p, first=False):
            q = 1 - p
            wait_idx(q)
            if not first:
                wait_out(q)
            fire_gather(q)
            wait_gather(p)
            compute(p)
            fire_out(j, p)
            fire_idx(jnp.minimum(j + 2, _NCH - 1), p)

        fire_idx(0, 0)
        wait_idx(0)
        fire_gather(0)
        fire_idx(1, 1)
        phase(0, 0, first=True)

        @pl.loop(1, _NCH - 1, step=2)
        def _(c):
            phase(c, 1)
            phase(c + 1, 0)

        wait_gather(1)
        compute(1)
        fire_out(_NCH - 1, 1)
        wait_idx(0)
        wait_out(0)
        wait_out(1)

    return k


_sc_embed = _build_sc_kernel()


def kernel(type, location, time, material, method_id, quantity,
           type_table, loc_table, time_table, mat_table, method_table,
           W_q, b_q):
    return _sc_embed(
        type, location, time, material, method_id, quantity,
        type_table, loc_table, time_table, mat_table, method_table,
        W_q, b_q)


# speedup vs baseline: 1.0407x; 1.0407x over previous
import dataclasses
import functools

import jax
import jax.numpy as jnp
from jax import lax
from jax.experimental import pallas as pl
from jax.experimental.pallas import tpu as pltpu
from jax.experimental.pallas import tpu_sc as plsc

_B, _L, _D = 4096, 200, 64
_NC, _NS = 2, 16
_NW = _NC * _NS
_NCH = _B // _NW
_NT, _NLOC, _NTIME, _NMETH = 9, 1000, 365, 100
_SPLITS = (0, 128)
_OR = _L * _D // 128


def _build_sc_kernel():
    mesh = plsc.VectorSubcoreMesh(core_axis_name="c", subcore_axis_name="s")
    cp = pltpu.CompilerParams()
    if "needs_layout_passes" in pltpu.CompilerParams.__dataclass_fields__:
        cp = dataclasses.replace(cp, needs_layout_passes=False)
    if "use_tc_tiling_on_sc" in pltpu.CompilerParams.__dataclass_fields__:
        cp = dataclasses.replace(cp, use_tc_tiling_on_sc=False)

    scratch = []
    for _ in range(2):
        scratch += [pltpu.VMEM((5, _L), jnp.int32)]
        scratch += [pltpu.VMEM((_L,), jnp.float32)]
        scratch += [pltpu.VMEM((_L, _D), jnp.float32)]
        scratch += [pltpu.VMEM((_L, _D), jnp.float32)]
        scratch += [pltpu.VMEM((_OR, 128), jnp.float32)]
    scratch += [
        pltpu.VMEM((_NT, _D), jnp.float32),
        pltpu.VMEM((_NTIME, _D), jnp.float32),
        pltpu.VMEM((_NMETH, _D), jnp.float32),
        pltpu.VMEM((_D,), jnp.float32),
        pltpu.VMEM((_D,), jnp.float32),
    ]
    scratch += [pltpu.SemaphoreType.DMA] * 6

    @functools.partial(
        pl.kernel,
        compiler_params=cp,
        out_type=jax.ShapeDtypeStruct((_B * _OR, 128), jnp.float32),
        mesh=mesh,
        scratch_types=scratch,
    )
    def k(ti_hbm, li_hbm, mi_hbm, ai_hbm, ei_hbm, q_hbm,
          ttab, ltab, titab, mtab, etab, wq_hbm, bq_hbm, out_hbm, *scr):
        stk = [scr[0], scr[5]]
        qv = [scr[1], scr[6]]
        gbuf = [scr[2], scr[7]]
        lbuf = [scr[3], scr[8]]
        ob = [scr[4], scr[9]]
        tres, mres, eres, wq_v, bq_v = scr[10:15]
        sem_idx, sem_g, sem_out = scr[15:17], scr[17:19], scr[19:21]

        idx_hbm = [ti_hbm, li_hbm, mi_hbm, ai_hbm, ei_hbm]

        wid = lax.axis_index("s") * _NC + lax.axis_index("c")
        pltpu.sync_copy(ttab, tres)
        pltpu.sync_copy(titab, mres)
        pltpu.sync_copy(etab, eres)
        pltpu.sync_copy(wq_hbm, wq_v)
        pltpu.sync_copy(bq_hbm, bq_v)
        wq = [wq_v[pl.ds(i * 16, 16)] for i in range(4)]
        bq = [bq_v[pl.ds(i * 16, 16)] for i in range(4)]

        @pl.loop(0, _NT)
        def _(r):
            for dd in range(4):
                sl = pl.ds(dd * 16, 16)
                tres[r, sl] = tres[r, sl] + bq[dd]

        def fire_idx(j, s):
            b = wid * _NCH + j
            for r, hbm in enumerate(idx_hbm):
                pltpu.async_copy(hbm.at[b], stk[s].at[r], sem_idx[s])
            pltpu.async_copy(q_hbm.at[b], qv[s], sem_idx[s])

        def wait_idx(s):
            pltpu.make_async_copy(ti_hbm.at[pl.ds(0, 5)], stk[s],
                                  sem_idx[s]).wait()
            pltpu.make_async_copy(q_hbm.at[0], qv[s], sem_idx[s]).wait()

        def fire_gather(s):
            bounds = _SPLITS + (_L,)
            for h, r0 in enumerate(_SPLITS):
                n = bounds[h + 1] - r0
                hs = pl.ds(r0, n)
                pltpu.async_copy(mtab.at[stk[s].at[3].at[hs]],
                                 gbuf[s].at[hs], sem_g[s])
                pltpu.async_copy(ltab.at[stk[s].at[1].at[hs]],
                                 lbuf[s].at[hs], sem_g[s])

        def wait_gather(s):
            pltpu.make_async_copy(mtab.at[stk[s].at[3]], gbuf[s],
                                  sem_g[s]).wait()
            pltpu.make_async_copy(ltab.at[stk[s].at[1]], lbuf[s],
                                  sem_g[s]).wait()

        def fire_out(j, s):
            b = wid * _NCH + j
            pltpu.async_copy(ob[s], out_hbm.at[pl.ds(b * _OR, _OR)],
                             sem_out[s])

        def wait_out(s):
            pltpu.make_async_copy(ob[s], out_hbm.at[pl.ds(0, _OR)],
                                  sem_out[s]).wait()

        def compute(s):
            sk = stk[s]
            qvv = qv[s]
            gb_ = gbuf[s]
            lb_ = lbuf[s]
            acc = ob[s]

            def do_group(gb, kks):
                sl16 = pl.ds(gb, 16)
                tvec = sk[0, sl16]
                mvec = sk[2, sl16]
                evec = sk[4, sl16]
                qvec = qvv[sl16]
                for kk in kks:
                    t = gb + kk
                    orow = (gb >> 1) + (kk >> 1)
                    ocol = (kk & 1) * 64
                    it, im = tvec[kk], mvec[kk]
                    ie = evec[kk]
                    q = lax.broadcast(qvec[kk], (16,))
                    for dd in range(4):
                        sl = pl.ds(dd * 16, 16)
                        s1 = gb_[t, sl] + tres[it, sl]
                        s2 = lb_[t, sl] + mres[im, sl]
                        s3 = eres[ie, sl] + q * wq[dd]
                        acc[orow, pl.ds(ocol + dd * 16, 16)] = (s1 + s2) + s3

            @pl.loop(0, (_L - 8) // 16)
            def _(g):
                do_group(g * 16, range(16))

            do_group(_L - 16, range(8, 16))

        def phase(j, p, first=False):
            q = 1 - p
            wait_idx(q)
            if not first:
                wait_out(q)
            fire_gather(q)
            wait_gather(p)
            compute(p)
            fire_out(j, p)
            fire_idx(jnp.minimum(j + 2, _NCH - 1), p)

        fire_idx(0, 0)
        wait_idx(0)
        fire_gather(0)
        fire_idx(1, 1)
        phase(0, 0, first=True)

        @pl.loop(1, _NCH - 1, step=2)
        def _(c):
            phase(c, 1)
            phase(c + 1, 0)

        wait_gather(1)
        compute(1)
        fire_out(_NCH - 1, 1)
        wait_idx(0)
        wait_out(0)
        wait_out(1)

    return k


_sc_embed = _build_sc_kernel()


def kernel(type, location, time, material, method_id, quantity,
           type_table, loc_table, time_table, mat_table, method_table,
           W_q, b_q):
    out = _sc_embed(
        type, location, time, material, method_id, quantity,
        type_table, loc_table, time_table, mat_table, method_table,
        W_q, b_q)
    return out.reshape(_B, _L, _D)


# speedup vs baseline: 1.2515x; 1.2026x over previous
import dataclasses
import functools

import jax
import jax.numpy as jnp
from jax import lax
from jax.experimental import pallas as pl
from jax.experimental.pallas import tpu as pltpu
from jax.experimental.pallas import tpu_sc as plsc

_B, _L, _D = 4096, 200, 64
_NC, _NS = 2, 16
_NW = _NC * _NS
_NCH = _B // _NW
_NT, _NLOC, _NTIME, _NMETH = 9, 1000, 365, 100
_SPLITS = (0, 128)
_OR = _L * _D // 128


def _build_sc_kernel():
    mesh = plsc.VectorSubcoreMesh(core_axis_name="c", subcore_axis_name="s")
    cp = pltpu.CompilerParams()
    if "needs_layout_passes" in pltpu.CompilerParams.__dataclass_fields__:
        cp = dataclasses.replace(cp, needs_layout_passes=False)
    if "use_tc_tiling_on_sc" in pltpu.CompilerParams.__dataclass_fields__:
        cp = dataclasses.replace(cp, use_tc_tiling_on_sc=False)

    scratch = []
    for _ in range(2):
        scratch += [pltpu.VMEM((5, _L), jnp.int32)]
        scratch += [pltpu.VMEM((_L,), jnp.float32)]
        scratch += [pltpu.VMEM((_L, _D), jnp.float32)]
        scratch += [pltpu.VMEM((_L, _D), jnp.float32)]
    scratch += [
        pltpu.VMEM((_NT, _D), jnp.float32),
        pltpu.VMEM((_NTIME, _D), jnp.float32),
        pltpu.VMEM((_NMETH, _D), jnp.float32),
        pltpu.VMEM((_D,), jnp.float32),
        pltpu.VMEM((_D,), jnp.float32),
    ]
    scratch += [pltpu.SemaphoreType.DMA] * 6

    @functools.partial(
        pl.kernel,
        compiler_params=cp,
        out_type=jax.ShapeDtypeStruct((_B * _L, 128), jnp.float32),
        mesh=mesh,
        scratch_types=scratch,
    )
    def k(ti_hbm, li_hbm, mi_hbm, ai_hbm, ei_hbm, q_hbm,
          ttab, ltab, titab, mtab, etab, wq_hbm, bq_hbm, out_hbm, *scr):
        stk = [scr[0], scr[4]]
        qv = [scr[1], scr[5]]
        gbuf = [scr[2], scr[6]]
        lbuf = [scr[3], scr[7]]
        tres, mres, eres, wq_v, bq_v = scr[8:13]
        sem_idx, sem_g, sem_out = scr[13:15], scr[15:17], scr[17:19]

        idx_hbm = [ti_hbm, li_hbm, mi_hbm, ai_hbm, ei_hbm]

        wid = lax.axis_index("s") * _NC + lax.axis_index("c")
        pltpu.sync_copy(ttab, tres)
        pltpu.sync_copy(titab, mres)
        pltpu.sync_copy(etab, eres)
        pltpu.sync_copy(wq_hbm, wq_v)
        pltpu.sync_copy(bq_hbm, bq_v)
        wq = [wq_v[pl.ds(i * 16, 16)] for i in range(4)]
        bq = [bq_v[pl.ds(i * 16, 16)] for i in range(4)]

        @pl.loop(0, _NT)
        def _(r):
            for dd in range(4):
                sl = pl.ds(dd * 16, 16)
                tres[r, sl] = tres[r, sl] + bq[dd]

        def fire_idx(j, s):
            b = wid * _NCH + j
            for r, hbm in enumerate(idx_hbm):
                pltpu.async_copy(hbm.at[b], stk[s].at[r], sem_idx[s])
            pltpu.async_copy(q_hbm.at[b], qv[s], sem_idx[s])

        def wait_idx(s):
            pltpu.make_async_copy(ti_hbm.at[pl.ds(0, 5)], stk[s],
                                  sem_idx[s]).wait()
            pltpu.make_async_copy(q_hbm.at[0], qv[s], sem_idx[s]).wait()

        def fire_gather(s):
            bounds = _SPLITS + (_L,)
            for h, r0 in enumerate(_SPLITS):
                n = bounds[h + 1] - r0
                hs = pl.ds(r0, n)
                pltpu.async_copy(mtab.at[stk[s].at[3].at[hs]],
                                 gbuf[s].at[hs], sem_g[s])
                pltpu.async_copy(ltab.at[stk[s].at[1].at[hs]],
                                 lbuf[s].at[hs], sem_g[s])

        def wait_gather(s):
            pltpu.make_async_copy(mtab.at[stk[s].at[3]], gbuf[s],
                                  sem_g[s]).wait()
            pltpu.make_async_copy(ltab.at[stk[s].at[1]], lbuf[s],
                                  sem_g[s]).wait()

        def fire_out(j, s):
            b = wid * _NCH + j
            pltpu.async_copy(
                gbuf[s], out_hbm.at[pl.ds(b * _L, _L), pl.ds(0, _D)],
                sem_out[s])

        def wait_out(s):
            pltpu.make_async_copy(
                gbuf[s], out_hbm.at[pl.ds(0, _L), pl.ds(0, _D)],
                sem_out[s]).wait()

        def compute(s):
            sk = stk[s]
            qvv = qv[s]
            gb_ = gbuf[s]
            lb_ = lbuf[s]
            acc = gb_

            def do_group(gb, kks):
                sl16 = pl.ds(gb, 16)
                tvec = sk[0, sl16]
                mvec = sk[2, sl16]
                evec = sk[4, sl16]
                qvec = qvv[sl16]
                for kk in kks:
                    t = gb + kk
                    it, im = tvec[kk], mvec[kk]
                    ie = evec[kk]
                    q = lax.broadcast(qvec[kk], (16,))
                    for dd in range(4):
                        sl = pl.ds(dd * 16, 16)
                        s1 = gb_[t, sl] + tres[it, sl]
                        s2 = lb_[t, sl] + mres[im, sl]
                        s3 = eres[ie, sl] + q * wq[dd]
                        acc[t, sl] = (s1 + s2) + s3

            @pl.loop(0, (_L - 8) // 16)
            def _(g):
                do_group(g * 16, range(16))

            do_group(_L - 16, range(8, 16))

        def phase(j, p, first=False):
            q = 1 - p
            wait_idx(q)
            if not first:
                wait_out(q)
            fire_gather(q)
            wait_gather(p)
            compute(p)
            fire_out(j, p)
            fire_idx(jnp.minimum(j + 2, _NCH - 1), p)

        fire_idx(0, 0)
        wait_idx(0)
        fire_gather(0)
        fire_idx(1, 1)
        phase(0, 0, first=True)

        @pl.loop(1, _NCH - 1, step=2)
        def _(c):
            phase(c, 1)
            phase(c + 1, 0)

        wait_gather(1)
        compute(1)
        fire_out(_NCH - 1, 1)
        wait_idx(0)
        wait_out(0)
        wait_out(1)

    return k


_sc_embed = _build_sc_kernel()


def kernel(type, location, time, material, method_id, quantity,
           type_table, loc_table, time_table, mat_table, method_table,
           W_q, b_q):
    out = _sc_embed(
        type, location, time, material, method_id, quantity,
        type_table, loc_table, time_table, mat_table, method_table,
        W_q, b_q)
    return out[:, :_D].reshape(_B, _L, _D)


# speedup vs baseline: 1.2807x; 1.0233x over previous
import dataclasses
import functools

import jax
import jax.numpy as jnp
from jax import lax
from jax.experimental import pallas as pl
from jax.experimental.pallas import tpu as pltpu
from jax.experimental.pallas import tpu_sc as plsc

_B, _L, _D = 4096, 200, 64
_NC, _NS = 2, 16
_NW = _NC * _NS
_NCH = _B // _NW
_NT, _NLOC, _NTIME, _NMETH = 9, 1000, 365, 100
_SPLITS = (0, 128)
_OR = _L * _D // 128


def _build_sc_kernel():
    mesh = plsc.VectorSubcoreMesh(core_axis_name="c", subcore_axis_name="s")
    cp = pltpu.CompilerParams()
    if "needs_layout_passes" in pltpu.CompilerParams.__dataclass_fields__:
        cp = dataclasses.replace(cp, needs_layout_passes=False)
    if "use_tc_tiling_on_sc" in pltpu.CompilerParams.__dataclass_fields__:
        cp = dataclasses.replace(cp, use_tc_tiling_on_sc=False)

    scratch = []
    for _ in range(2):
        scratch += [pltpu.VMEM((5, _L), jnp.int32)]
        scratch += [pltpu.VMEM((_L,), jnp.float32)]
        scratch += [pltpu.VMEM((_L, _D), jnp.float32)]
        scratch += [pltpu.VMEM((_L, _D), jnp.float32)]
    scratch += [
        pltpu.VMEM((_NT, _D), jnp.float32),
        pltpu.VMEM((_NTIME, _D), jnp.float32),
        pltpu.VMEM((_NMETH, _D), jnp.float32),
        pltpu.VMEM((_D,), jnp.float32),
        pltpu.VMEM((_D,), jnp.float32),
    ]
    scratch += [pltpu.SemaphoreType.DMA] * 6

    @functools.partial(
        pl.kernel,
        compiler_params=cp,
        out_type=jax.ShapeDtypeStruct((_B * _L, 128), jnp.float32),
        mesh=mesh,
        scratch_types=scratch,
    )
    def k(ti_hbm, li_hbm, mi_hbm, ai_hbm, ei_hbm, q_hbm,
          ttab, ltab, titab, mtab, etab, wq_hbm, bq_hbm, out_hbm, *scr):
        stk = [scr[0], scr[4]]
        qv = [scr[1], scr[5]]
        gbuf = [scr[2], scr[6]]
        lbuf = [scr[3], scr[7]]
        tres, mres, eres, wq_v, bq_v = scr[8:13]
        sem_idx, sem_g, sem_out = scr[13:15], scr[15:17], scr[17:19]

        idx_hbm = [ti_hbm, li_hbm, mi_hbm, ai_hbm, ei_hbm]

        wid = lax.axis_index("s") * _NC + lax.axis_index("c")
        pltpu.sync_copy(ttab, tres)
        pltpu.sync_copy(titab, mres)
        pltpu.sync_copy(etab, eres)
        pltpu.sync_copy(wq_hbm, wq_v)
        pltpu.sync_copy(bq_hbm, bq_v)
        wq = [wq_v[pl.ds(i * 16, 16)] for i in range(4)]
        bq = [bq_v[pl.ds(i * 16, 16)] for i in range(4)]

        @pl.loop(0, _NT)
        def _(r):
            for dd in range(4):
                sl = pl.ds(dd * 16, 16)
                tres[r, sl] = tres[r, sl] + bq[dd]

        def fire_idx(j, s):
            b = wid * _NCH + j
            for r, hbm in enumerate(idx_hbm):
                pltpu.async_copy(hbm.at[b], stk[s].at[r], sem_idx[s])
            pltpu.async_copy(q_hbm.at[b], qv[s], sem_idx[s])

        def wait_idx(s):
            pltpu.make_async_copy(ti_hbm.at[pl.ds(0, 5)], stk[s],
                                  sem_idx[s]).wait()
            pltpu.make_async_copy(q_hbm.at[0], qv[s], sem_idx[s]).wait()

        def fire_gather(s):
            bounds = _SPLITS + (_L,)
            for h, r0 in enumerate(_SPLITS):
                n = bounds[h + 1] - r0
                hs = pl.ds(r0, n)
                pltpu.async_copy(mtab.at[stk[s].at[3].at[hs]],
                                 gbuf[s].at[hs], sem_g[s])
                pltpu.async_copy(ltab.at[stk[s].at[1].at[hs]],
                                 lbuf[s].at[hs], sem_g[s])

        def wait_gather(s):
            pltpu.make_async_copy(mtab.at[stk[s].at[3]], gbuf[s],
                                  sem_g[s]).wait()
            pltpu.make_async_copy(ltab.at[stk[s].at[1]], lbuf[s],
                                  sem_g[s]).wait()

        def fire_out(j, s):
            b = wid * _NCH + j
            pltpu.async_copy(
                gbuf[s], out_hbm.at[pl.ds(b * _L, _L), pl.ds(0, _D)],
                sem_out[s])

        def wait_out(s):
            pltpu.make_async_copy(
                gbuf[s], out_hbm.at[pl.ds(0, _L), pl.ds(0, _D)],
                sem_out[s]).wait()

        def compute(s):
            sk = stk[s]
            qvv = qv[s]
            gb_ = gbuf[s]
            lb_ = lbuf[s]
            acc = gb_

            def do_group(gb, kks):
                sl16 = pl.ds(gb, 16)
                tvec = sk[0, sl16]
                mvec = sk[2, sl16]
                evec = sk[4, sl16]
                qvec = qvv[sl16]
                for kk in kks:
                    t = gb + kk
                    it, im = tvec[kk], mvec[kk]
                    ie = evec[kk]
                    q = lax.broadcast(qvec[kk], (16,))
                    for dd in range(4):
                        sl = pl.ds(dd * 16, 16)
                        s1 = gb_[t, sl] + tres[it, sl]
                        s2 = lb_[t, sl] + mres[im, sl]
                        s3 = eres[ie, sl] + q * wq[dd]
                        acc[t, sl] = (s1 + s2) + s3

            @pl.loop(0, (_L - 8) // 16, unroll=2)
            def _(g):
                do_group(g * 16, range(16))

            do_group(_L - 16, range(8, 16))

        def phase(j, p, first=False):
            q = 1 - p
            wait_idx(q)
            if not first:
                wait_out(q)
            fire_gather(q)
            wait_gather(p)
            compute(p)
            fire_out(j, p)
            fire_idx(jnp.minimum(j + 2, _NCH - 1), p)

        fire_idx(0, 0)
        wait_idx(0)
        fire_gather(0)
        fire_idx(1, 1)
        phase(0, 0, first=True)

        @pl.loop(1, _NCH - 1, step=2)
        def _(c):
            phase(c, 1)
            phase(c + 1, 0)

        wait_gather(1)
        compute(1)
        fire_out(_NCH - 1, 1)
        wait_idx(0)
        wait_out(0)
        wait_out(1)

    return k


_sc_embed = _build_sc_kernel()


def kernel(type, location, time, material, method_id, quantity,
           type_table, loc_table, time_table, mat_table, method_table,
           W_q, b_q):
    out = _sc_embed(
        type, location, time, material, method_id, quantity,
        type_table, loc_table, time_table, mat_table, method_table,
        W_q, b_q)
    return out[:, :_D].reshape(_B, _L, _D)
